# Initial kernel scaffold; baseline (speedup 1.0000x reference)
import jax
import jax.numpy as jnp
from jax.experimental import pallas as pl


def kernel(token_ids, phrase_ids, fast_table, slow_table):
    raise NotImplementedError("write your pallas kernel here")


# speedup vs baseline: 3.1604x; 3.1604x over previous
import functools

import jax
import jax.numpy as jnp
from jax import lax
from jax.experimental import pallas as pl
from jax.experimental.pallas import tpu as pltpu
from jax.experimental.pallas import tpu_sc as plsc

_D = 64
_TOK_V = 100000
_PHR_V = 8192
_BLK = 512
_FAST_BLOCKS = 196
_SLOW_BASE = _FAST_BLOCKS * _BLK
_COMB_ROWS = _SLOW_BASE + _PHR_V


def _normalize_body(fast_ref, slow_ref, out_ref):
    i = pl.program_id(0)

    @pl.when(i < _FAST_BLOCKS)
    def _():
        x = fast_ref[...]
        n = jnp.sqrt(jnp.sum(x * x, axis=-1, keepdims=True))
        out_ref[...] = x / (n + 1e-8)

    @pl.when(i >= _FAST_BLOCKS)
    def _():
        x = slow_ref[...]
        n = jnp.sqrt(jnp.sum(x * x, axis=-1, keepdims=True))
        out_ref[...] = x / (n + 1e-8)


def _normalize_tables(fast_table, slow_table):
    grid = _COMB_ROWS // _BLK
    return pl.pallas_call(
        _normalize_body,
        grid=(grid,),
        in_specs=[
            pl.BlockSpec((_BLK, _D), lambda i: (jnp.minimum(i, _FAST_BLOCKS - 1), 0)),
            pl.BlockSpec((_BLK, _D), lambda i: (jnp.clip(i - _FAST_BLOCKS, 0, 15), 0)),
        ],
        out_specs=pl.BlockSpec((_BLK, _D), lambda i: (i, 0)),
        out_shape=jax.ShapeDtypeStruct((_COMB_ROWS, _D), jnp.float32),
    )(fast_table, slow_table)


def _make_gather(n_rows: int):
    info = plsc.get_sparse_core_info()
    nc, ns = info.num_cores, info.num_subcores
    nw = nc * ns
    per_w = n_rows // nw
    chunk = 128
    n_iter = per_w // chunk

    mesh = plsc.VectorSubcoreMesh(core_axis_name="c", subcore_axis_name="s")

    @functools.partial(
        pl.kernel,
        mesh=mesh,
        out_type=jax.ShapeDtypeStruct((n_rows, _D), jnp.float32),
        compiler_params=pltpu.CompilerParams(use_tc_tiling_on_sc=False),
        scratch_types=[
            pltpu.VMEM((chunk,), jnp.int32),
            pltpu.VMEM((chunk, _D), jnp.float32),
            pltpu.SemaphoreType.DMA,
        ],
    )
    def gather_k(table_hbm, idx_hbm, out_hbm, idx_v, rows_v, sem):
        wid = lax.axis_index("s") * nc + lax.axis_index("c")
        base0 = wid * per_w

        def body(j, carry):
            base = base0 + j * chunk
            pltpu.sync_copy(idx_hbm.at[pl.ds(base, chunk)], idx_v)
            pltpu.async_copy(table_hbm.at[idx_v], rows_v, sem).wait()
            pltpu.sync_copy(rows_v, out_hbm.at[pl.ds(base, chunk)])
            return carry

        lax.fori_loop(0, n_iter, body, 0)

    return gather_k


def kernel(token_ids, phrase_ids, fast_table, slow_table):
    b, l = token_ids.shape
    n = b * l
    tok = token_ids.reshape(-1).astype(jnp.int32)
    phr = phrase_ids.reshape(-1).astype(jnp.int32) + jnp.int32(_SLOW_BASE)
    idx2 = jnp.stack([tok, phr], axis=-1).reshape(-1)

    comb = _normalize_tables(fast_table, slow_table)
    out = _make_gather(2 * n)(comb, idx2)
    return out.reshape(b, l, 2 * _D)


# speedup vs baseline: 3.7123x; 1.1746x over previous
import functools

import jax
import jax.numpy as jnp
from jax import lax
from jax.experimental import pallas as pl
from jax.experimental.pallas import tpu as pltpu
from jax.experimental.pallas import tpu_sc as plsc

_D = 64
_TOK_V = 100000
_PHR_V = 8192
_BLK = 512
_FAST_BLOCKS = 196
_SLOW_BASE = _FAST_BLOCKS * _BLK
_COMB_ROWS = _SLOW_BASE + _PHR_V


def _normalize_body(fast_ref, slow_ref, out_ref):
    i = pl.program_id(0)

    @pl.when(i < _FAST_BLOCKS)
    def _():
        x = fast_ref[...]
        n = jnp.sqrt(jnp.sum(x * x, axis=-1, keepdims=True))
        out_ref[...] = x / (n + 1e-8)

    @pl.when(i >= _FAST_BLOCKS)
    def _():
        x = slow_ref[...]
        n = jnp.sqrt(jnp.sum(x * x, axis=-1, keepdims=True))
        out_ref[...] = x / (n + 1e-8)


def _normalize_tables(fast_table, slow_table):
    grid = _COMB_ROWS // _BLK
    return pl.pallas_call(
        _normalize_body,
        grid=(grid,),
        in_specs=[
            pl.BlockSpec((_BLK, _D), lambda i: (jnp.minimum(i, _FAST_BLOCKS - 1), 0)),
            pl.BlockSpec((_BLK, _D), lambda i: (jnp.clip(i - _FAST_BLOCKS, 0, 15), 0)),
        ],
        out_specs=pl.BlockSpec((_BLK, _D), lambda i: (i, 0)),
        out_shape=jax.ShapeDtypeStruct((_COMB_ROWS, _D), jnp.float32),
    )(fast_table, slow_table)


def _make_gather(n_rows: int):
    info = plsc.get_sparse_core_info()
    nc, ns = info.num_cores, info.num_subcores
    nw = nc * ns
    per_w = n_rows // nw
    chunk = 128
    cpg = 5
    grp = chunk * cpg
    n_grp = per_w // grp

    mesh = plsc.VectorSubcoreMesh(core_axis_name="c", subcore_axis_name="s")

    @functools.partial(
        pl.kernel,
        mesh=mesh,
        out_type=jax.ShapeDtypeStruct((n_rows, _D), jnp.float32),
        compiler_params=pltpu.CompilerParams(use_tc_tiling_on_sc=False),
        scratch_types=[
            pltpu.VMEM((per_w,), jnp.int32),
            pltpu.VMEM((grp, _D), jnp.float32),
            pltpu.VMEM((grp, _D), jnp.float32),
            pltpu.SemaphoreType.DMA,
            pltpu.SemaphoreType.DMA,
            pltpu.SemaphoreType.DMA,
            pltpu.SemaphoreType.DMA,
        ],
    )
    def gather_k(table_hbm, idx_hbm, out_hbm, idx_v, buf_a, buf_b,
                 gsem_a, gsem_b, wsem_a, wsem_b):
        wid = lax.axis_index("s") * nc + lax.axis_index("c")
        base0 = wid * per_w
        pltpu.sync_copy(idx_hbm.at[pl.ds(base0, per_w)], idx_v)

        def run_group(t, g, buf, gsem, wsem):
            @pl.when(t > 0)
            def _():
                pltpu.make_async_copy(
                    buf, out_hbm.at[pl.ds(base0, grp)], wsem).wait()
            l0 = g * grp
            cps = [
                pltpu.async_copy(
                    table_hbm.at[idx_v.at[pl.ds(l0 + b * chunk, chunk)]],
                    buf.at[pl.ds(b * chunk, chunk)],
                    gsem,
                )
                for b in range(cpg)
            ]
            for cp in cps:
                cp.wait()
            pltpu.async_copy(buf, out_hbm.at[pl.ds(base0 + l0, grp)], wsem)

        def body(t, carry):
            run_group(t, 2 * t, buf_a, gsem_a, wsem_a)
            run_group(t, 2 * t + 1, buf_b, gsem_b, wsem_b)
            return carry

        lax.fori_loop(0, n_grp // 2, body, 0)
        pltpu.make_async_copy(buf_a, out_hbm.at[pl.ds(base0, grp)], wsem_a).wait()
        pltpu.make_async_copy(buf_b, out_hbm.at[pl.ds(base0, grp)], wsem_b).wait()

    return gather_k


def kernel(token_ids, phrase_ids, fast_table, slow_table):
    b, l = token_ids.shape
    n = b * l
    tok = token_ids.reshape(-1).astype(jnp.int32)
    phr = phrase_ids.reshape(-1).astype(jnp.int32) + jnp.int32(_SLOW_BASE)
    idx2 = jnp.stack([tok, phr], axis=-1).reshape(-1)

    comb = _normalize_tables(fast_table, slow_table)
    out = _make_gather(2 * n)(comb, idx2)
    return out.reshape(b, l, 2 * _D)


# speedup vs baseline: 7.3111x; 1.9694x over previous
import functools

import jax
import jax.numpy as jnp
from jax import lax
from jax.experimental import pallas as pl
from jax.experimental.pallas import tpu as pltpu
from jax.experimental.pallas import tpu_sc as plsc

_D = 64
_TOK_V = 100000
_PHR_V = 8192
_PBLK = 1024
_FAST_PBLOCKS = 49
_SLOW_BASE = 2 * _FAST_PBLOCKS * _PBLK
_COMB_PROWS = _FAST_PBLOCKS * _PBLK + _PHR_V // 2
_COMB_ROWS = 2 * _COMB_PROWS


def _normalize_body(fast_ref, slow_ref, out_ref):
    i = pl.program_id(0)

    def norm_pair(x):
        xl, xr = x[:, :_D], x[:, _D:]
        nl = jnp.sqrt(jnp.sum(xl * xl, axis=-1, keepdims=True))
        nr = jnp.sqrt(jnp.sum(xr * xr, axis=-1, keepdims=True))
        return jnp.concatenate([xl / (nl + 1e-8), xr / (nr + 1e-8)], axis=-1)

    @pl.when(i < _FAST_PBLOCKS)
    def _():
        out_ref[...] = norm_pair(fast_ref[...])

    @pl.when(i >= _FAST_PBLOCKS)
    def _():
        out_ref[...] = norm_pair(slow_ref[...])


def _normalize_tables(fast2, slow2):
    grid = _COMB_PROWS // _PBLK
    return pl.pallas_call(
        _normalize_body,
        grid=(grid,),
        in_specs=[
            pl.BlockSpec((_PBLK, 2 * _D),
                         lambda i: (jnp.minimum(i, _FAST_PBLOCKS - 1), 0)),
            pl.BlockSpec((_PBLK, 2 * _D),
                         lambda i: (jnp.clip(i - _FAST_PBLOCKS, 0, 3), 0)),
        ],
        out_specs=pl.BlockSpec((_PBLK, 2 * _D), lambda i: (i, 0)),
        out_shape=jax.ShapeDtypeStruct((_COMB_PROWS, 2 * _D), jnp.float32),
    )(fast2, slow2)


def _make_gather(n_rows: int):
    info = plsc.get_sparse_core_info()
    nc, ns = info.num_cores, info.num_subcores
    nw = nc * ns
    per_w = n_rows // nw
    chunk = 128
    cpg = 5
    grp = chunk * cpg
    n_grp = per_w // grp

    mesh = plsc.VectorSubcoreMesh(core_axis_name="c", subcore_axis_name="s")

    @functools.partial(
        pl.kernel,
        mesh=mesh,
        out_type=jax.ShapeDtypeStruct((n_rows, _D), jnp.float32),
        compiler_params=pltpu.CompilerParams(use_tc_tiling_on_sc=False),
        scratch_types=[
            pltpu.VMEM((per_w,), jnp.int32),
            pltpu.VMEM((grp, _D), jnp.float32),
            pltpu.VMEM((grp, _D), jnp.float32),
            pltpu.SemaphoreType.DMA,
            pltpu.SemaphoreType.DMA,
            pltpu.SemaphoreType.DMA,
            pltpu.SemaphoreType.DMA,
        ],
    )
    def gather_k(table_hbm, idx_hbm, out_hbm, idx_v, buf_a, buf_b,
                 gsem_a, gsem_b, wsem_a, wsem_b):
        wid = lax.axis_index("s") * nc + lax.axis_index("c")
        base0 = wid * per_w
        pltpu.sync_copy(idx_hbm.at[pl.ds(base0, per_w)], idx_v)

        def run_group(t, g, buf, gsem, wsem):
            @pl.when(t > 0)
            def _():
                pltpu.make_async_copy(
                    buf, out_hbm.at[pl.ds(base0, grp)], wsem).wait()
            l0 = g * grp
            cps = [
                pltpu.async_copy(
                    table_hbm.at[idx_v.at[pl.ds(l0 + b * chunk, chunk)]],
                    buf.at[pl.ds(b * chunk, chunk)],
                    gsem,
                )
                for b in range(cpg)
            ]
            for cp in cps:
                cp.wait()
            pltpu.async_copy(buf, out_hbm.at[pl.ds(base0 + l0, grp)], wsem)

        def body(t, carry):
            run_group(t, 2 * t, buf_a, gsem_a, wsem_a)
            run_group(t, 2 * t + 1, buf_b, gsem_b, wsem_b)
            return carry

        lax.fori_loop(0, n_grp // 2, body, 0)
        pltpu.make_async_copy(buf_a, out_hbm.at[pl.ds(base0, grp)], wsem_a).wait()
        pltpu.make_async_copy(buf_b, out_hbm.at[pl.ds(base0, grp)], wsem_b).wait()

    return gather_k


def kernel(token_ids, phrase_ids, fast_table, slow_table):
    b, l = token_ids.shape
    n = b * l
    tok = token_ids.T.reshape(-1).astype(jnp.int32)
    phr = phrase_ids.T.reshape(-1).astype(jnp.int32) + jnp.int32(_SLOW_BASE)
    idx2 = jnp.stack([tok, phr], axis=-1).reshape(-1)

    comb = _normalize_tables(
        fast_table.reshape(_TOK_V // 2, 2 * _D),
        slow_table.reshape(_PHR_V // 2, 2 * _D),
    ).reshape(_COMB_ROWS, _D)
    out = _make_gather(2 * n)(comb, idx2)
    return out.reshape(l, b, 2 * _D).transpose(1, 0, 2)
